# baseline (speedup 1.0000x reference)
import functools
import math

import jax
import jax.numpy as jnp
from jax import lax
from jax.experimental import pallas as pl
from jax.experimental.pallas import tpu as pltpu
from jax.experimental.pallas import tpu_sc as plsc

D_EMB = 128
MAX_MODEL_LEN = 2048
B = 16
L = 2048

NC = 2
NS = 16
LANES = 16
NW = NC * NS
ROWS = B * L
PER_W = ROWS // NW
CHUNK = 128
N_CHUNKS = PER_W // CHUNK
INV_SQRT_D = 1.0 / math.sqrt(D_EMB)


def _precompute_pe_t_scaled():
    pos_arg = jnp.arange(0, MAX_MODEL_LEN, dtype=jnp.float32)
    dim_arg = (10000.0 ** ((jnp.arange(0, D_EMB, dtype=jnp.float32) / 2.0)
                           / D_EMB)).reshape(-1, 1)
    pe = pos_arg / dim_arg
    pe = pe.at[::2].set(jnp.sin(pe[::2]))
    pe = pe.at[1::2].set(jnp.cos(pe[1::2]))
    return pe.T * INV_SQRT_D


def _tec_body(tokens_hbm, pe_hbm, nv_hbm, emb_hbm, out_hbm,
              idx_v, rows_v, pe_v, nv_v, sem):
    cid = lax.axis_index("c")
    sid = lax.axis_index("s")
    wid = cid * NS + sid
    base = wid * PER_W
    pe_base = lax.rem(wid, 2) * PER_W

    pltpu.sync_copy(nv_hbm, nv_v)
    nv_w = nv_v[pl.ds(wid, LANES)][0]

    zero = jnp.zeros((LANES,), jnp.float32)

    for k in range(N_CHUNKS):
        row0 = base + k * CHUNK
        nvk = jnp.clip(nv_w - k * CHUNK, 0, CHUNK)

        @pl.when(nvk > 0)
        def _():
            pltpu.sync_copy(tokens_hbm.at[pl.ds(row0, CHUNK)], idx_v)
            gather = pltpu.async_copy(emb_hbm.at[idx_v], rows_v, sem)
            pltpu.sync_copy(pe_hbm.at[pl.ds(pe_base + k * CHUNK, CHUNK), :],
                            pe_v)
            gather.wait()

            def fma_body(r, carry):
                for c in range(D_EMB // LANES):
                    sl = pl.ds(c * LANES, LANES)
                    rows_v[r, sl] = rows_v[r, sl] * INV_SQRT_D + pe_v[r, sl]
                return carry

            lax.fori_loop(0, nvk, fma_body, 0)

        def zero_body(r, carry):
            for c in range(D_EMB // LANES):
                rows_v[r, pl.ds(c * LANES, LANES)] = zero
            return carry

        lax.fori_loop(nvk, CHUNK, zero_body, 0)

        pltpu.sync_copy(rows_v, out_hbm.at[pl.ds(row0, CHUNK), :])


@jax.jit
def _run(tokens_flat, pe_t, nv, emb_matrix):
    mesh = plsc.VectorSubcoreMesh(core_axis_name="c", subcore_axis_name="s",
                                  num_cores=NC, num_subcores=NS)
    out = pl.kernel(
        _tec_body,
        out_type=jax.ShapeDtypeStruct((ROWS, D_EMB), jnp.float32),
        mesh=mesh,
        scratch_types=[
            pltpu.VMEM((CHUNK,), jnp.int32),
            pltpu.VMEM((CHUNK, D_EMB), jnp.float32),
            pltpu.VMEM((CHUNK, D_EMB), jnp.float32),
            pltpu.VMEM((NW + LANES,), jnp.int32),
            pltpu.SemaphoreType.DMA,
        ],
    )(tokens_flat, pe_t, nv, emb_matrix)
    return out.reshape(B, L, D_EMB)


def kernel(tokens, lengths, emb_matrix):
    tokens_flat = tokens.reshape(ROWS).astype(jnp.int32)
    pe_t = _precompute_pe_t_scaled()
    wid = jnp.arange(NW, dtype=jnp.int32)
    seq = wid // (L // PER_W)
    l0 = (wid % (L // PER_W)) * PER_W
    nv = jnp.clip(lengths.astype(jnp.int32)[seq] - l0, 0, PER_W)
    nv = jnp.concatenate([nv, jnp.zeros((LANES,), jnp.int32)])
    return _run(tokens_flat, pe_t, nv, emb_matrix)


# speedup vs baseline: 1.6687x; 1.6687x over previous
import math

import jax
import jax.numpy as jnp
from jax import lax
from jax.experimental import pallas as pl
from jax.experimental.pallas import tpu as pltpu
from jax.experimental.pallas import tpu_sc as plsc

D_EMB = 128
MAX_MODEL_LEN = 2048
B = 16
L = 2048

NC = 2
NS = 16
LANES = 16
NW = NC * NS
ROWS = B * L
PER_W = ROWS // NW
CHUNK = 256
N_CHUNKS = PER_W // CHUNK
IDX_BLK = 128
N_SUB = CHUNK // IDX_BLK
GROUPS = D_EMB // LANES
INV_SQRT_D = 1.0 / math.sqrt(D_EMB)


def _precompute_pe_t():
    pos_arg = jnp.arange(0, MAX_MODEL_LEN, dtype=jnp.float32)
    dim_arg = (10000.0 ** ((jnp.arange(0, D_EMB, dtype=jnp.float32) / 2.0)
                           / D_EMB)).reshape(-1, 1)
    pe = pos_arg / dim_arg
    pe = pe.at[::2].set(jnp.sin(pe[::2]))
    pe = pe.at[1::2].set(jnp.cos(pe[1::2]))
    return pe.T


def _tec_body(tokens_hbm, pe_hbm, nv_hbm, emb_hbm, zeros_hbm, out_hbm,
              idx_v, rows0, rows1, zbuf, nv_v,
              pe_sem, g_sem, wb_sem0, wb_sem1, z_sem):
    cid = lax.axis_index("c")
    sid = lax.axis_index("s")
    wid = cid * NS + sid
    base = wid * PER_W
    pe0 = lax.rem(wid, 2) * PER_W

    pltpu.sync_copy(nv_hbm, nv_v)
    nv_w = nv_v[pl.ds(wid, LANES)][0]
    pltpu.sync_copy(zeros_hbm, zbuf)
    pltpu.sync_copy(
        tokens_hbm.at[pl.ds(wid * (PER_W // IDX_BLK), PER_W // IDX_BLK), :],
        idx_v)

    bufs = [rows0, rows1]
    wsems = [wb_sem0, wb_sem1]
    nvks = [jnp.clip(nv_w - k * CHUNK, 0, CHUNK) for k in range(N_CHUNKS)]
    zero_vec = jnp.zeros((LANES,), jnp.float32)

    def pe_desc(k):
        return pltpu.make_async_copy(
            pe_hbm.at[pl.ds(pe0 + k * CHUNK, CHUNK), :], bufs[k % 2], pe_sem)

    def g_desc(k, j):
        return pltpu.make_async_copy(
            emb_hbm.at[idx_v.at[k * N_SUB + j]],
            bufs[k % 2].at[pl.ds(j * IDX_BLK, IDX_BLK), :], g_sem)

    def wb_desc(k):
        return pltpu.make_async_copy(
            bufs[k % 2], out_hbm.at[pl.ds(base + k * CHUNK, CHUNK), :],
            wsems[k % 2])

    def zwb_desc(k):
        return pltpu.make_async_copy(
            zbuf, out_hbm.at[pl.ds(base + k * CHUNK, CHUNK), :], z_sem)

    def issue_pe(k):
        @pl.when(nvks[k] > 0)
        def _():
            pe_desc(k).start()

    def issue_gather(k):
        @pl.when(nvks[k] > 0)
        def _():
            pe_desc(k).wait()
            buf = bufs[k % 2]
            pltpu.async_copy(emb_hbm.at[idx_v.at[k * N_SUB]],
                             buf.at[pl.ds(0, IDX_BLK), :], g_sem, add=True)
            pltpu.async_copy(emb_hbm.at[idx_v.at[k * N_SUB + 1]],
                             buf.at[pl.ds(IDX_BLK, IDX_BLK), :], g_sem,
                             add=True)

    def finish(k):
        nvk = nvks[k]
        buf = bufs[k % 2]

        @pl.when(nvk > 0)
        def _():
            g_desc(k, 0).wait()
            g_desc(k, 1).wait()

            def scale_body(r, carry):
                for c in range(GROUPS):
                    sl = pl.ds(c * LANES, LANES)
                    buf[r, sl] = buf[r, sl] * INV_SQRT_D
                return carry

            lax.fori_loop(0, nvk, scale_body, 0)

            def tail_body(r, carry):
                for c in range(GROUPS):
                    buf[r, pl.ds(c * LANES, LANES)] = zero_vec
                return carry

            lax.fori_loop(nvk, CHUNK, tail_body, 0)
            wb_desc(k).start()

        @pl.when(nvk <= 0)
        def _():
            zwb_desc(k).start()

    def retire_wb(k):
        @pl.when(nvks[k] > 0)
        def _():
            wb_desc(k).wait()

    for k in range(N_CHUNKS):
        if k >= 2:
            retire_wb(k - 2)
        issue_pe(k)
        if k >= 1:
            finish(k - 1)
        issue_gather(k)
    finish(N_CHUNKS - 1)
    for k in (N_CHUNKS - 2, N_CHUNKS - 1):
        retire_wb(k)
    for k in range(N_CHUNKS):
        @pl.when(nvks[k] <= 0)
        def _(k=k):
            zwb_desc(k).wait()


@jax.jit
def _run(tokens_2d, pe_t, nv, emb_matrix, zeros):
    mesh = plsc.VectorSubcoreMesh(core_axis_name="c", subcore_axis_name="s",
                                  num_cores=NC, num_subcores=NS)
    out = pl.kernel(
        _tec_body,
        out_type=jax.ShapeDtypeStruct((ROWS, D_EMB), jnp.float32),
        mesh=mesh,
        scratch_types=[
            pltpu.VMEM((PER_W // IDX_BLK, IDX_BLK), jnp.int32),
            pltpu.VMEM((CHUNK, D_EMB), jnp.float32),
            pltpu.VMEM((CHUNK, D_EMB), jnp.float32),
            pltpu.VMEM((CHUNK, D_EMB), jnp.float32),
            pltpu.VMEM((NW + LANES,), jnp.int32),
            pltpu.SemaphoreType.DMA,
            pltpu.SemaphoreType.DMA,
            pltpu.SemaphoreType.DMA,
            pltpu.SemaphoreType.DMA,
            pltpu.SemaphoreType.DMA,
        ],
    )(tokens_2d, pe_t, nv, emb_matrix, zeros)
    return out.reshape(B, L, D_EMB)


def kernel(tokens, lengths, emb_matrix):
    tokens_2d = tokens.reshape(ROWS // IDX_BLK, IDX_BLK).astype(jnp.int32)
    pe_t = _precompute_pe_t()
    wid = jnp.arange(NW, dtype=jnp.int32)
    seq = wid // (L // PER_W)
    l0 = (wid % (L // PER_W)) * PER_W
    nv = jnp.clip(lengths.astype(jnp.int32)[seq] - l0, 0, PER_W)
    nv = jnp.concatenate([nv, jnp.zeros((LANES,), jnp.int32)])
    zeros = jnp.zeros((CHUNK, D_EMB), jnp.float32)
    return _run(tokens_2d, pe_t, nv, emb_matrix, zeros)
